# baseline (speedup 1.0000x reference)
import jax
import jax.numpy as jnp
from jax.experimental import pallas as pl

_B, _D = 16384, 64
_BLK = 512
_EPS = 1e-12


def _body(gu_ref, gi_ref, gis_ref, xui_ref, guo_ref, gio_ref, giso_ref):
    gu = gu_ref[...]
    gi = gi_ref[...]
    gis = gis_ref[...]
    guo_ref[...] = gu
    gio_ref[...] = gi
    giso_ref[...] = gis
    n = jnp.sqrt(jnp.sum(gis * gis, axis=1, keepdims=True))
    gis_n = gis / jnp.maximum(n, _EPS)
    final = gi + gis_n
    xui_ref[...] = jnp.sum(gu * final, axis=1)


def kernel(gu, gi, gis):
    grid = (_B // _BLK,)
    in_spec = pl.BlockSpec((_BLK, _D), lambda i: (i, 0))
    out_specs = (
        pl.BlockSpec((_BLK,), lambda i: (i,)),
        in_spec,
        in_spec,
        in_spec,
    )
    xui, guo, gio, giso = pl.pallas_call(
        _body,
        grid=grid,
        in_specs=[in_spec, in_spec, in_spec],
        out_specs=out_specs,
        out_shape=(
            jax.ShapeDtypeStruct((_B,), jnp.float32),
            jax.ShapeDtypeStruct((_B, _D), jnp.float32),
            jax.ShapeDtypeStruct((_B, _D), jnp.float32),
            jax.ShapeDtypeStruct((_B, _D), jnp.float32),
        ),
    )(gu, gi, gis)
    return (xui, guo, gio, giso)
